# baseline (speedup 1.0000x reference)
import functools

import jax
import jax.numpy as jnp
from jax import lax
from jax.experimental import pallas as pl
from jax.experimental.pallas import tpu as pltpu
from jax.experimental.pallas import tpu_sc as plsc

NVARS = 512
BATCH = 4096
L = 16
NVEC = NVARS // L


def _build_permute():
    info = plsc.get_sparse_core_info()
    nc, ns = info.num_cores, info.num_subcores
    nw = nc * ns
    rows_per_w = BATCH // nw
    chunk = 64
    n_chunks = rows_per_w // chunk

    mesh = plsc.VectorSubcoreMesh(core_axis_name="c", subcore_axis_name="s")

    @functools.partial(
        pl.kernel,
        mesh=mesh,
        out_type=jax.ShapeDtypeStruct((BATCH, NVARS), jnp.float32),
        compiler_params=pltpu.CompilerParams(needs_layout_passes=False),
        scratch_types=[
            pltpu.VMEM((NVARS,), jnp.int32),
            pltpu.VMEM((chunk, NVARS), jnp.float32),
            pltpu.VMEM((chunk, NVARS), jnp.float32),
            pltpu.VMEM((chunk, NVARS), jnp.float32),
            pltpu.SemaphoreType.DMA,
            pltpu.SemaphoreType.DMA,
            pltpu.SemaphoreType.DMA,
        ],
    )
    def permute(x_hbm, perm_hbm, out_hbm, idx_v, in0, in1, ob,
                idx_sem, in_sem, out_sem):
        wid = lax.axis_index("s") * nc + lax.axis_index("c")
        base = wid * rows_per_w

        idx_h = pltpu.async_copy(perm_hbm, idx_v, idx_sem)
        in_h = [
            pltpu.async_copy(x_hbm.at[pl.ds(base, chunk)], in0, in_sem),
            pltpu.async_copy(x_hbm.at[pl.ds(base + chunk, chunk)], in1,
                             in_sem),
        ]

        idx_h.wait()
        jgroup = 16

        def compute(in_b, out_b):
            for g in range(NVEC // jgroup):
                colsg = [idx_v[pl.ds((g * jgroup + jj) * L, L)]
                         for jj in range(jgroup)]

                @plsc.parallel_loop(0, chunk, 1, unroll=4)
                def _row(r, _colsg=colsg, _g=g):
                    rsplat = jnp.full((L,), r, jnp.int32)
                    for jj in range(jgroup):
                        gv = plsc.load_gather(in_b, [rsplat, _colsg[jj]])
                        out_b[r, pl.ds((_g * jgroup + jj) * L, L)] = gv

        in_h[0].wait()
        compute(in0, ob)
        oh0 = pltpu.async_copy(ob, out_hbm.at[pl.ds(base, chunk)], out_sem)

        in_h[1].wait()
        compute(in1, in0)
        oh1 = pltpu.async_copy(in0, out_hbm.at[pl.ds(base + chunk, chunk)],
                               out_sem)
        oh0.wait()
        oh1.wait()

    return permute


_permute = _build_permute()


def kernel(x, context, perm):
    y = _permute(x, perm.astype(jnp.int32))
    return y, 0


# speedup vs baseline: 1.0851x; 1.0851x over previous
import functools

import jax
import jax.numpy as jnp
from jax import lax
from jax.experimental import pallas as pl
from jax.experimental.pallas import tpu as pltpu
from jax.experimental.pallas import tpu_sc as plsc

NVARS = 512
BATCH = 4096
L = 16
NVEC = NVARS // L


def _build_permute():
    info = plsc.get_sparse_core_info()
    nc, ns = info.num_cores, info.num_subcores
    nw = nc * ns
    rows_per_w = BATCH // nw
    chunk = 64
    n_chunks = rows_per_w // chunk

    mesh = plsc.VectorSubcoreMesh(core_axis_name="c", subcore_axis_name="s")

    @functools.partial(
        pl.kernel,
        mesh=mesh,
        out_type=jax.ShapeDtypeStruct((BATCH, NVARS), jnp.float32),
        compiler_params=pltpu.CompilerParams(needs_layout_passes=False),
        scratch_types=[
            pltpu.VMEM((NVARS,), jnp.int32),
            pltpu.VMEM((chunk, NVARS), jnp.float32),
            pltpu.VMEM((chunk, NVARS), jnp.float32),
            pltpu.VMEM((chunk, NVARS), jnp.float32),
            pltpu.SemaphoreType.DMA,
            pltpu.SemaphoreType.DMA,
            pltpu.SemaphoreType.DMA,
        ],
    )
    def permute(x_hbm, perm_hbm, out_hbm, idx_v, in0, in1, ob,
                idx_sem, in_sem, out_sem):
        wid = lax.axis_index("s") * nc + lax.axis_index("c")
        base = wid * rows_per_w

        idx_h = pltpu.async_copy(perm_hbm, idx_v, idx_sem)
        in_h = [
            pltpu.async_copy(x_hbm.at[pl.ds(base, chunk)], in0, in_sem),
            pltpu.async_copy(x_hbm.at[pl.ds(base + chunk, chunk)], in1,
                             in_sem),
        ]

        idx_h.wait()
        jgroup = 32

        def compute(in_b, out_b):
            for g in range(NVEC // jgroup):
                colsg = [idx_v[pl.ds((g * jgroup + jj) * L, L)]
                         for jj in range(jgroup)]

                @plsc.parallel_loop(0, chunk, 1, unroll=1)
                def _row(r, _colsg=colsg, _g=g):
                    rsplat = jnp.full((L,), r, jnp.int32)
                    for jj in range(jgroup):
                        gv = plsc.load_gather(in_b, [rsplat, _colsg[jj]])
                        out_b[r, pl.ds((_g * jgroup + jj) * L, L)] = gv

        in_h[0].wait()
        compute(in0, ob)
        oh0 = pltpu.async_copy(ob, out_hbm.at[pl.ds(base, chunk)], out_sem)

        in_h[1].wait()
        compute(in1, in0)
        oh1 = pltpu.async_copy(in0, out_hbm.at[pl.ds(base + chunk, chunk)],
                               out_sem)
        oh0.wait()
        oh1.wait()

    return permute


_permute = _build_permute()


def kernel(x, context, perm):
    y = _permute(x, perm.astype(jnp.int32))
    return y, 0
